# Initial kernel scaffold; baseline (speedup 1.0000x reference)
import jax
import jax.numpy as jnp
from jax.experimental import pallas as pl


def kernel(value, table):
    raise NotImplementedError("write your pallas kernel here")


# speedup vs baseline: 4.8080x; 4.8080x over previous
import functools

import jax
import jax.numpy as jnp
from jax import lax
from jax.experimental import pallas as pl
from jax.experimental.pallas import tpu as pltpu
from jax.experimental.pallas import tpu_sc as plsc

_NC, _NS = 2, 16
_NW = _NC * _NS
_CW = 128
_K = 8


@functools.lru_cache(maxsize=None)
def _build(B, V, D):
    CHUNK = _K * _CW
    assert B % (_NW * CHUNK) == 0
    b_per_w = B // _NW
    n_steps = b_per_w // CHUNK
    rows_per_w = b_per_w // _CW

    mesh = plsc.VectorSubcoreMesh(core_axis_name="c", subcore_axis_name="s")

    @functools.partial(
        pl.kernel,
        out_type=jax.ShapeDtypeStruct((B, D), jnp.float32),
        mesh=mesh,
        compiler_params=pltpu.CompilerParams(use_tc_tiling_on_sc=False),
        scratch_types=[
            pltpu.VMEM((_K, _CW), jnp.int32),
            pltpu.VMEM((CHUNK, D), jnp.float32),
            pltpu.SemaphoreType.DMA,
        ],
    )
    def gather_kernel(idx_hbm, table_hbm, out_hbm, idx_v, rows_v, sem):
        wid = lax.axis_index("s") * _NC + lax.axis_index("c")
        idx_row0 = wid * rows_per_w
        out_base = wid * b_per_w

        def step(g, carry):
            pltpu.sync_copy(idx_hbm.at[pl.ds(idx_row0 + g * _K, _K)], idx_v)
            copies = [
                pltpu.async_copy(
                    table_hbm.at[idx_v.at[j]],
                    rows_v.at[pl.ds(j * _CW, _CW)],
                    sem,
                )
                for j in range(_K)
            ]
            for c in copies:
                c.wait()
            pltpu.sync_copy(
                rows_v, out_hbm.at[pl.ds(out_base + g * CHUNK, CHUNK)]
            )
            return carry

        lax.fori_loop(0, n_steps, step, 0)

    return gather_kernel


def kernel(value, table):
    bsz, seq = value.shape
    V, D = table.shape
    B = bsz * seq
    idx2d = value.astype(jnp.int32).reshape(B // _CW, _CW)
    out = _build(B, V, D)(idx2d, table)
    return out.reshape(bsz, seq, D)


# speedup vs baseline: 4.8086x; 1.0001x over previous
import functools

import jax
import jax.numpy as jnp
from jax import lax
from jax.experimental import pallas as pl
from jax.experimental.pallas import tpu as pltpu
from jax.experimental.pallas import tpu_sc as plsc

_NC, _NS = 2, 16
_NW = _NC * _NS
_CW = 1024
_K = 1


@functools.lru_cache(maxsize=None)
def _build(B, V, D):
    CHUNK = _K * _CW
    assert B % (_NW * CHUNK) == 0
    b_per_w = B // _NW
    n_steps = b_per_w // CHUNK
    rows_per_w = b_per_w // _CW

    mesh = plsc.VectorSubcoreMesh(core_axis_name="c", subcore_axis_name="s")

    @functools.partial(
        pl.kernel,
        out_type=jax.ShapeDtypeStruct((B, D), jnp.float32),
        mesh=mesh,
        compiler_params=pltpu.CompilerParams(use_tc_tiling_on_sc=False),
        scratch_types=[
            pltpu.VMEM((_K, _CW), jnp.int32),
            pltpu.VMEM((CHUNK, D), jnp.float32),
            pltpu.SemaphoreType.DMA,
        ],
    )
    def gather_kernel(idx_hbm, table_hbm, out_hbm, idx_v, rows_v, sem):
        wid = lax.axis_index("s") * _NC + lax.axis_index("c")
        idx_row0 = wid * rows_per_w
        out_base = wid * b_per_w

        def step(g, carry):
            pltpu.sync_copy(idx_hbm.at[pl.ds(idx_row0 + g * _K, _K)], idx_v)
            copies = [
                pltpu.async_copy(
                    table_hbm.at[idx_v.at[j]],
                    rows_v.at[pl.ds(j * _CW, _CW)],
                    sem,
                )
                for j in range(_K)
            ]
            for c in copies:
                c.wait()
            pltpu.sync_copy(
                rows_v, out_hbm.at[pl.ds(out_base + g * CHUNK, CHUNK)]
            )
            return carry

        lax.fori_loop(0, n_steps, step, 0)

    return gather_kernel


def kernel(value, table):
    bsz, seq = value.shape
    V, D = table.shape
    B = bsz * seq
    idx2d = value.astype(jnp.int32).reshape(B // _CW, _CW)
    out = _build(B, V, D)(idx2d, table)
    return out.reshape(bsz, seq, D)


# speedup vs baseline: 5.0478x; 1.0497x over previous
import functools

import jax
import jax.numpy as jnp
from jax import lax
from jax.experimental import pallas as pl
from jax.experimental.pallas import tpu as pltpu
from jax.experimental.pallas import tpu_sc as plsc

_NC, _NS = 2, 16
_NW = _NC * _NS
_CHUNK = 1024


@functools.lru_cache(maxsize=None)
def _build(B, V, D):
    assert B % (_NW * _CHUNK) == 0
    b_per_w = B // _NW
    n_steps = b_per_w // _CHUNK
    mesh = plsc.VectorSubcoreMesh(core_axis_name="c", subcore_axis_name="s")

    @functools.partial(
        pl.kernel,
        out_type=jax.ShapeDtypeStruct((B, D), jnp.float32),
        mesh=mesh,
        compiler_params=pltpu.CompilerParams(use_tc_tiling_on_sc=False),
        scratch_types=[
            pltpu.VMEM((2, _CHUNK), jnp.int32),
            pltpu.VMEM((2, _CHUNK, D), jnp.float32),
            pltpu.SemaphoreType.DMA,
            pltpu.SemaphoreType.DMA,
            pltpu.SemaphoreType.DMA,
            pltpu.SemaphoreType.DMA,
            pltpu.SemaphoreType.DMA,
            pltpu.SemaphoreType.DMA,
        ],
    )
    def gather_kernel(idx_hbm, table_hbm, out_hbm, idx_v, rows_v,
                      si0, si1, sg0, sg1, so0, so1):
        sem_idx, sem_g, sem_o = [si0, si1], [sg0, sg1], [so0, so1]
        wid = lax.axis_index("s") * _NC + lax.axis_index("c")
        row0 = wid * n_steps
        out_base = wid * b_per_w

        def idx_load(g, b):
            return pltpu.make_async_copy(
                idx_hbm.at[row0 + g], idx_v.at[b], sem_idx[b])

        def gather(b):
            return pltpu.make_async_copy(
                table_hbm.at[idx_v.at[b]], rows_v.at[b], sem_g[b])

        def out_write(g, b):
            return pltpu.make_async_copy(
                rows_v.at[b],
                out_hbm.at[pl.ds(out_base + g * _CHUNK, _CHUNK)],
                sem_o[b])

        idx_load(0, 0).start()
        idx_load(1, 1).start()

        def outer(G, carry):
            for b in (0, 1):
                g = G * 2 + b
                idx_load(g, b).wait()
                @pl.when(G > 0)
                def _():
                    out_write(g - 2, b).wait()
                gather(b).start()
                prev_ready = (G > 0) if b == 0 else (G >= 0)
                @pl.when(prev_ready)
                def _():
                    gather(1 - b).wait()
                    out_write(g - 1, 1 - b).start()
                if b == 0:
                    @pl.when(G > 0)
                    def _():
                        idx_load(g + 1, 1).start()
                else:
                    @pl.when(G < n_steps // 2 - 1)
                    def _():
                        idx_load(g + 1, 0).start()
            return carry

        lax.fori_loop(0, n_steps // 2, outer, 0, unroll=False)

        last = n_steps - 1
        b_last = last % 2
        gather(b_last).wait()
        out_write(last, b_last).start()
        out_write(last - 1, 1 - b_last).wait()
        out_write(last, b_last).wait()

    return gather_kernel


def kernel(value, table):
    bsz, seq = value.shape
    V, D = table.shape
    B = bsz * seq
    idx2d = value.astype(jnp.int32).reshape(B // _CHUNK, _CHUNK)
    out = _build(B, V, D)(idx2d, table)
    return out.reshape(bsz, seq, D)


# speedup vs baseline: 5.0494x; 1.0003x over previous
import functools

import jax
import jax.numpy as jnp
from jax import lax
from jax.experimental import pallas as pl
from jax.experimental.pallas import tpu as pltpu
from jax.experimental.pallas import tpu_sc as plsc

_NC, _NS = 2, 16
_NW = _NC * _NS
_CHUNK = 1600


@functools.lru_cache(maxsize=None)
def _build(B, V, D):
    assert B % (_NW * _CHUNK) == 0
    b_per_w = B // _NW
    n_steps = b_per_w // _CHUNK
    assert n_steps % 2 == 0
    mesh = plsc.VectorSubcoreMesh(core_axis_name="c", subcore_axis_name="s")

    @functools.partial(
        pl.kernel,
        out_type=jax.ShapeDtypeStruct((B, D), jnp.float32),
        mesh=mesh,
        compiler_params=pltpu.CompilerParams(use_tc_tiling_on_sc=False),
        scratch_types=[
            pltpu.VMEM((2, _CHUNK), jnp.int32),
            pltpu.VMEM((2, _CHUNK, D), jnp.float32),
            pltpu.SemaphoreType.DMA,
            pltpu.SemaphoreType.DMA,
            pltpu.SemaphoreType.DMA,
            pltpu.SemaphoreType.DMA,
            pltpu.SemaphoreType.DMA,
            pltpu.SemaphoreType.DMA,
        ],
    )
    def gather_kernel(idx_hbm, table_hbm, out_hbm, idx_v, rows_v,
                      si0, si1, sg0, sg1, so0, so1):
        sem_idx, sem_g, sem_o = [si0, si1], [sg0, sg1], [so0, so1]
        wid = lax.axis_index("s") * _NC + lax.axis_index("c")
        row0 = wid * n_steps
        out_base = wid * b_per_w

        def idx_load(g, b):
            return pltpu.make_async_copy(
                idx_hbm.at[row0 + g], idx_v.at[b], sem_idx[b])

        def gather(b):
            return pltpu.make_async_copy(
                table_hbm.at[idx_v.at[b]], rows_v.at[b], sem_g[b])

        def out_write(g, b):
            return pltpu.make_async_copy(
                rows_v.at[b],
                out_hbm.at[pl.ds(out_base + g * _CHUNK, _CHUNK)],
                sem_o[b])

        idx_load(0, 0).start()
        idx_load(1, 1).start()

        def outer(G, carry):
            for b in (0, 1):
                g = G * 2 + b
                idx_load(g, b).wait()
                @pl.when(G > 0)
                def _():
                    out_write(g - 2, b).wait()
                gather(b).start()
                prev_ready = (G > 0) if b == 0 else (G >= 0)
                @pl.when(prev_ready)
                def _():
                    gather(1 - b).wait()
                    out_write(g - 1, 1 - b).start()
                if b == 0:
                    @pl.when(G > 0)
                    def _():
                        idx_load(g + 1, 1).start()
                else:
                    @pl.when(G < n_steps // 2 - 1)
                    def _():
                        idx_load(g + 1, 0).start()
            return carry

        lax.fori_loop(0, n_steps // 2, outer, 0, unroll=False)

        last = n_steps - 1
        b_last = last % 2
        gather(b_last).wait()
        out_write(last, b_last).start()
        out_write(last - 1, 1 - b_last).wait()
        out_write(last, b_last).wait()

    return gather_kernel


def kernel(value, table):
    bsz, seq = value.shape
    V, D = table.shape
    B = bsz * seq
    idx2d = value.astype(jnp.int32).reshape(B // _CHUNK, _CHUNK)
    out = _build(B, V, D)(idx2d, table)
    return out.reshape(bsz, seq, D)


# speedup vs baseline: 5.0502x; 1.0002x over previous
import functools

import jax
import jax.numpy as jnp
from jax import lax
from jax.experimental import pallas as pl
from jax.experimental.pallas import tpu as pltpu
from jax.experimental.pallas import tpu_sc as plsc

_NC, _NS = 2, 16
_NW = _NC * _NS
_CHUNK = 1024


@functools.lru_cache(maxsize=None)
def _build(B, V, D):
    assert B % (_NW * _CHUNK) == 0
    b_per_w = B // _NW
    n_steps = b_per_w // _CHUNK
    assert n_steps % 2 == 0
    mesh = plsc.VectorSubcoreMesh(core_axis_name="c", subcore_axis_name="s")

    @functools.partial(
        pl.kernel,
        out_type=jax.ShapeDtypeStruct((B, D), jnp.float32),
        mesh=mesh,
        compiler_params=pltpu.CompilerParams(use_tc_tiling_on_sc=False),
        scratch_types=[
            pltpu.VMEM((2, _CHUNK), jnp.int32),
            pltpu.VMEM((2, _CHUNK, D), jnp.float32),
            pltpu.SemaphoreType.DMA,
            pltpu.SemaphoreType.DMA,
            pltpu.SemaphoreType.DMA,
            pltpu.SemaphoreType.DMA,
            pltpu.SemaphoreType.DMA,
            pltpu.SemaphoreType.DMA,
        ],
    )
    def gather_kernel(idx_hbm, table_hbm, out_hbm, idx_v, rows_v,
                      si0, si1, sg0, sg1, so0, so1):
        sem_idx, sem_g, sem_o = [si0, si1], [sg0, sg1], [so0, so1]
        wid = lax.axis_index("s") * _NC + lax.axis_index("c")
        row0 = wid * n_steps
        out_base = wid * b_per_w

        def idx_load(g, b):
            return pltpu.make_async_copy(
                idx_hbm.at[row0 + g], idx_v.at[b], sem_idx[b])

        def gather(b):
            return pltpu.make_async_copy(
                table_hbm.at[idx_v.at[b]], rows_v.at[b], sem_g[b])

        def out_write(g, b):
            return pltpu.make_async_copy(
                rows_v.at[b],
                out_hbm.at[pl.ds(out_base + g * _CHUNK, _CHUNK)],
                sem_o[b])

        idx_load(0, 0).start()
        idx_load(1, 1).start()

        def outer(G, carry):
            for b in (0, 1):
                g = G * 2 + b
                idx_load(g, b).wait()
                @pl.when(G > 0)
                def _():
                    out_write(g - 2, b).wait()
                gather(b).start()
                prev_ready = (G > 0) if b == 0 else (G >= 0)
                @pl.when(prev_ready)
                def _():
                    gather(1 - b).wait()
                    out_write(g - 1, 1 - b).start()
                if b == 0:
                    @pl.when(G > 0)
                    def _():
                        idx_load(g + 1, 1).start()
                else:
                    @pl.when(G < n_steps // 2 - 1)
                    def _():
                        idx_load(g + 1, 0).start()
            return carry

        lax.fori_loop(0, n_steps // 2, outer, 0, unroll=False)

        last = n_steps - 1
        b_last = last % 2
        gather(b_last).wait()
        out_write(last, b_last).start()
        out_write(last - 1, 1 - b_last).wait()
        out_write(last, b_last).wait()

    return gather_kernel


def kernel(value, table):
    bsz, seq = value.shape
    V, D = table.shape
    B = bsz * seq
    idx2d = value.astype(jnp.int32).reshape(B // _CHUNK, _CHUNK)
    out = _build(B, V, D)(idx2d, table)
    return out.reshape(bsz, seq, D)
